# baseline (speedup 1.0000x reference)
import jax
import jax.numpy as jnp
from jax.experimental import pallas as pl

_T = 4096


def _split3(W):
    h1 = W.astype(jnp.bfloat16)
    r1 = W - h1.astype(jnp.float32)
    h2 = r1.astype(jnp.bfloat16)
    h3 = (r1 - h2.astype(jnp.float32)).astype(jnp.bfloat16)
    return jnp.concatenate([h1, h2, h3], axis=1)


def _rvq_body(x_ref, c0, c1, c2, c3, n0, n1, n2, n3,
              h0, h1, h2, idx_ref, dist_ref):
    x = x_ref[:]
    T, D = x.shape
    cbs = (c0, c1, c2, c3)
    cns = (n0, n1, n2, n3)
    his = (h0, h1, h2, None)
    K = c0.shape[0]
    iota_f = jax.lax.broadcasted_iota(
        jnp.int32, (1, K), 1).astype(jnp.float32)
    r = x
    rn = jnp.sum(r * r, axis=1, keepdims=True)
    idxs = []
    dsqs = []
    for s in range(4):
        Sm2 = jax.lax.dot_general(
            r, cbs[s][:], dimension_numbers=(((1,), (1,)), ((), ())),
            preferred_element_type=jnp.float32)
        sc = cns[s][:] + Sm2
        m = jnp.min(sc, axis=1, keepdims=True)
        i_f = jnp.min(jnp.where(sc == m, iota_f, jnp.float32(K)),
                      axis=1, keepdims=True)
        idxs.append(i_f.astype(jnp.int32))
        rn = rn + m
        dsqs.append(rn)
        if s < 3:
            onehot = (iota_f == i_f).astype(jnp.bfloat16)
            gg = jax.lax.dot_general(
                onehot, his[s][:], dimension_numbers=(((1,), (0,)), ((), ())),
                preferred_element_type=jnp.float32)
            r = r - ((gg[:, :D] + gg[:, D : 2 * D]) + gg[:, 2 * D :])
    idx_ref[:] = jnp.concatenate(idxs, axis=1)
    dist_ref[:] = jnp.sqrt(
        jnp.maximum(jnp.concatenate(dsqs, axis=1), 1e-12))


def kernel(X, cb0, cb1, cb2, cb3, return_dist):
    N, D = X.shape
    K = cb0.shape[0]
    grid = N // _T
    cbs = (cb0, cb1, cb2, cb3)
    cbs_m2 = tuple(-2.0 * W for W in cbs)
    cns = tuple(jnp.sum(W * W, axis=1)[None, :] for W in cbs)
    splits = tuple(_split3(W) for W in cbs[:3])
    wspec = pl.BlockSpec((K, D), lambda i: (0, 0))
    nspec = pl.BlockSpec((1, K), lambda i: (0, 0))
    hspec = pl.BlockSpec((K, 3 * D), lambda i: (0, 0))
    idx, dist = pl.pallas_call(
        _rvq_body,
        grid=(grid,),
        in_specs=[pl.BlockSpec((_T, D), lambda i: (i, 0))]
        + [wspec] * 4 + [nspec] * 4 + [hspec] * 3,
        out_specs=[pl.BlockSpec((_T, 4), lambda i: (i, 0)),
                   pl.BlockSpec((_T, 4), lambda i: (i, 0))],
        out_shape=[jax.ShapeDtypeStruct((N, 4), jnp.int32),
                   jax.ShapeDtypeStruct((N, 4), jnp.float32)],
    )(X, *cbs_m2, *cns, *splits)
    gate = jnp.asarray(return_dist, jnp.float32)
    return idx, dist * gate
